# baseline (device time: 16304 ns/iter reference)
import jax
import jax.numpy as jnp
from jax import lax
from jax.experimental import pallas as pl
from jax.experimental.pallas import tpu as pltpu

CHUNK_ROWS = [16, 24, 40, 48, 48, 40, 24, 16]
N_CHUNKS = len(CHUNK_ROWS)
CHUNK_OFFS = [sum(CHUNK_ROWS[:i]) for i in range(N_CHUNKS)]


def kernel(A, B):
    m, k = A.shape
    _, n = B.shape
    half = m // 2
    assert sum(CHUNK_ROWS) == half

    def body(a_ref, b_ref, out_ref,
             acc_ref, xrecv_ref,
             x_send_sems, x_recv_sems, y_send_sems, y_recv_sems):
        my_x = lax.axis_index("x")
        my_y = lax.axis_index("y")
        x_nbr = (1 - my_x, my_y)
        y_nbr = (my_x, 1 - my_y)

        barrier_sem = pltpu.get_barrier_semaphore()
        for nbr in (x_nbr, y_nbr):
            pl.semaphore_signal(
                barrier_sem, inc=1, device_id=nbr,
                device_id_type=pl.DeviceIdType.MESH,
            )
        pl.semaphore_wait(barrier_sem, 2)

        my_base = my_y * half

        x_rdmas = []
        for c in range(N_CHUNKS):
            rows = pl.ds(CHUNK_OFFS[c], CHUNK_ROWS[c])
            acc_ref[rows, :] = jnp.dot(
                a_ref[pl.ds(my_base + CHUNK_OFFS[c], CHUNK_ROWS[c]), :],
                b_ref[:, :],
                preferred_element_type=jnp.float32,
            )
            rdma = pltpu.make_async_remote_copy(
                src_ref=acc_ref.at[rows, :],
                dst_ref=xrecv_ref.at[rows, :],
                send_sem=x_send_sems.at[c],
                recv_sem=x_recv_sems.at[c],
                device_id=x_nbr,
                device_id_type=pl.DeviceIdType.MESH,
            )
            rdma.start()
            x_rdmas.append(rdma)

        y_rdmas = []
        for c in range(N_CHUNKS):
            rows = pl.ds(CHUNK_OFFS[c], CHUNK_ROWS[c])
            out_rows = pl.ds(my_base + CHUNK_OFFS[c], CHUNK_ROWS[c])
            x_rdmas[c].wait_recv()
            out_ref[out_rows, :] = acc_ref[rows, :] + xrecv_ref[rows, :]
            rdma = pltpu.make_async_remote_copy(
                src_ref=out_ref.at[out_rows, :],
                dst_ref=out_ref.at[out_rows, :],
                send_sem=y_send_sems.at[c],
                recv_sem=y_recv_sems.at[c],
                device_id=y_nbr,
                device_id_type=pl.DeviceIdType.MESH,
            )
            rdma.start()
            y_rdmas.append(rdma)

        for c in range(N_CHUNKS):
            y_rdmas[c].wait_recv()
        for c in range(N_CHUNKS):
            x_rdmas[c].wait_send()
            y_rdmas[c].wait_send()

    return pl.pallas_call(
        body,
        out_shape=jax.ShapeDtypeStruct((m, n), jnp.float32),
        in_specs=[
            pl.BlockSpec(memory_space=pltpu.VMEM),
            pl.BlockSpec(memory_space=pltpu.VMEM),
        ],
        out_specs=pl.BlockSpec(memory_space=pltpu.VMEM),
        scratch_shapes=[
            pltpu.VMEM((half, n), jnp.float32),
            pltpu.VMEM((half, n), jnp.float32),
            pltpu.SemaphoreType.DMA((N_CHUNKS,)),
            pltpu.SemaphoreType.DMA((N_CHUNKS,)),
            pltpu.SemaphoreType.DMA((N_CHUNKS,)),
            pltpu.SemaphoreType.DMA((N_CHUNKS,)),
        ],
        compiler_params=pltpu.CompilerParams(collective_id=0),
    )(A, B)


# device time: 16077 ns/iter; 1.0141x vs baseline; 1.0141x over previous
import jax
import jax.numpy as jnp
from jax import lax
from jax.experimental import pallas as pl
from jax.experimental.pallas import tpu as pltpu

CHUNK_ROWS = [32] * 8
N_CHUNKS = len(CHUNK_ROWS)
CHUNK_OFFS = [sum(CHUNK_ROWS[:i]) for i in range(N_CHUNKS)]


def kernel(A, B):
    m, k = A.shape
    _, n = B.shape
    half = m // 2
    assert sum(CHUNK_ROWS) == half

    def body(a_ref, b_ref, out_ref,
             acc_ref, xrecv_ref,
             x_send_sems, x_recv_sems, y_send_sems, y_recv_sems):
        my_x = lax.axis_index("x")
        my_y = lax.axis_index("y")
        x_nbr = (1 - my_x, my_y)
        y_nbr = (my_x, 1 - my_y)

        my_base = my_y * half

        acc_ref[:, :] = jnp.dot(
            a_ref[pl.ds(my_base, half), :], b_ref[:, :],
            preferred_element_type=jnp.float32,
        )

        barrier_sem = pltpu.get_barrier_semaphore()
        for nbr in (x_nbr, y_nbr):
            pl.semaphore_signal(
                barrier_sem, inc=1, device_id=nbr,
                device_id_type=pl.DeviceIdType.MESH,
            )
        pl.semaphore_wait(barrier_sem, 2)

        x_rdmas = []
        for c in range(N_CHUNKS):
            rows = pl.ds(CHUNK_OFFS[c], CHUNK_ROWS[c])
            rdma = pltpu.make_async_remote_copy(
                src_ref=acc_ref.at[rows, :],
                dst_ref=xrecv_ref.at[rows, :],
                send_sem=x_send_sems.at[c],
                recv_sem=x_recv_sems.at[c],
                device_id=x_nbr,
                device_id_type=pl.DeviceIdType.MESH,
            )
            rdma.start()
            x_rdmas.append(rdma)

        y_rdmas = []
        for c in range(N_CHUNKS):
            rows = pl.ds(CHUNK_OFFS[c], CHUNK_ROWS[c])
            out_rows = pl.ds(my_base + CHUNK_OFFS[c], CHUNK_ROWS[c])
            x_rdmas[c].wait_recv()
            out_ref[out_rows, :] = acc_ref[rows, :] + xrecv_ref[rows, :]
            rdma = pltpu.make_async_remote_copy(
                src_ref=out_ref.at[out_rows, :],
                dst_ref=out_ref.at[out_rows, :],
                send_sem=y_send_sems.at[c],
                recv_sem=y_recv_sems.at[c],
                device_id=y_nbr,
                device_id_type=pl.DeviceIdType.MESH,
            )
            rdma.start()
            y_rdmas.append(rdma)

        for c in range(N_CHUNKS):
            y_rdmas[c].wait_recv()
        for c in range(N_CHUNKS):
            x_rdmas[c].wait_send()
            y_rdmas[c].wait_send()

    return pl.pallas_call(
        body,
        out_shape=jax.ShapeDtypeStruct((m, n), jnp.float32),
        in_specs=[
            pl.BlockSpec(memory_space=pltpu.VMEM),
            pl.BlockSpec(memory_space=pltpu.VMEM),
        ],
        out_specs=pl.BlockSpec(memory_space=pltpu.VMEM),
        scratch_shapes=[
            pltpu.VMEM((half, n), jnp.float32),
            pltpu.VMEM((half, n), jnp.float32),
            pltpu.SemaphoreType.DMA((N_CHUNKS,)),
            pltpu.SemaphoreType.DMA((N_CHUNKS,)),
            pltpu.SemaphoreType.DMA((N_CHUNKS,)),
            pltpu.SemaphoreType.DMA((N_CHUNKS,)),
        ],
        compiler_params=pltpu.CompilerParams(collective_id=0),
    )(A, B)


# device time: 16065 ns/iter; 1.0149x vs baseline; 1.0007x over previous
import jax
import jax.numpy as jnp
from jax import lax
from jax.experimental import pallas as pl
from jax.experimental.pallas import tpu as pltpu

CHUNK_ROWS = [16] * 16
N_CHUNKS = len(CHUNK_ROWS)
CHUNK_OFFS = [sum(CHUNK_ROWS[:i]) for i in range(N_CHUNKS)]


def kernel(A, B):
    m, k = A.shape
    _, n = B.shape
    half = m // 2
    assert sum(CHUNK_ROWS) == half

    def body(a_ref, b_ref, out_ref,
             acc_ref, xrecv_ref,
             x_send_sems, x_recv_sems, y_send_sems, y_recv_sems):
        my_x = lax.axis_index("x")
        my_y = lax.axis_index("y")
        x_nbr = (1 - my_x, my_y)
        y_nbr = (my_x, 1 - my_y)

        my_base = my_y * half

        acc_ref[:, :] = jnp.dot(
            a_ref[pl.ds(my_base, half), :], b_ref[:, :],
            preferred_element_type=jnp.float32,
        )

        barrier_sem = pltpu.get_barrier_semaphore()
        for nbr in (x_nbr, y_nbr):
            pl.semaphore_signal(
                barrier_sem, inc=1, device_id=nbr,
                device_id_type=pl.DeviceIdType.MESH,
            )
        pl.semaphore_wait(barrier_sem, 2)

        x_rdmas = []
        for c in range(N_CHUNKS):
            rows = pl.ds(CHUNK_OFFS[c], CHUNK_ROWS[c])
            rdma = pltpu.make_async_remote_copy(
                src_ref=acc_ref.at[rows, :],
                dst_ref=xrecv_ref.at[rows, :],
                send_sem=x_send_sems.at[c],
                recv_sem=x_recv_sems.at[c],
                device_id=x_nbr,
                device_id_type=pl.DeviceIdType.MESH,
            )
            rdma.start()
            x_rdmas.append(rdma)

        y_rdmas = []
        for c in range(N_CHUNKS):
            rows = pl.ds(CHUNK_OFFS[c], CHUNK_ROWS[c])
            out_rows = pl.ds(my_base + CHUNK_OFFS[c], CHUNK_ROWS[c])
            x_rdmas[c].wait_recv()
            out_ref[out_rows, :] = acc_ref[rows, :] + xrecv_ref[rows, :]
            rdma = pltpu.make_async_remote_copy(
                src_ref=out_ref.at[out_rows, :],
                dst_ref=out_ref.at[out_rows, :],
                send_sem=y_send_sems.at[c],
                recv_sem=y_recv_sems.at[c],
                device_id=y_nbr,
                device_id_type=pl.DeviceIdType.MESH,
            )
            rdma.start()
            y_rdmas.append(rdma)

        for c in range(N_CHUNKS):
            y_rdmas[c].wait_recv()
        for c in range(N_CHUNKS):
            x_rdmas[c].wait_send()
            y_rdmas[c].wait_send()

    return pl.pallas_call(
        body,
        out_shape=jax.ShapeDtypeStruct((m, n), jnp.float32),
        in_specs=[
            pl.BlockSpec(memory_space=pltpu.VMEM),
            pl.BlockSpec(memory_space=pltpu.VMEM),
        ],
        out_specs=pl.BlockSpec(memory_space=pltpu.VMEM),
        scratch_shapes=[
            pltpu.VMEM((half, n), jnp.float32),
            pltpu.VMEM((half, n), jnp.float32),
            pltpu.SemaphoreType.DMA((N_CHUNKS,)),
            pltpu.SemaphoreType.DMA((N_CHUNKS,)),
            pltpu.SemaphoreType.DMA((N_CHUNKS,)),
            pltpu.SemaphoreType.DMA((N_CHUNKS,)),
        ],
        compiler_params=pltpu.CompilerParams(collective_id=0),
    )(A, B)


# device time: 15817 ns/iter; 1.0308x vs baseline; 1.0157x over previous
import jax
import jax.numpy as jnp
from jax import lax
from jax.experimental import pallas as pl
from jax.experimental.pallas import tpu as pltpu

CHUNK_ROWS = [16] * 16
N_CHUNKS = len(CHUNK_ROWS)
CHUNK_OFFS = [sum(CHUNK_ROWS[:i]) for i in range(N_CHUNKS)]


def kernel(A, B):
    m, k = A.shape
    _, n = B.shape
    half = m // 2
    assert sum(CHUNK_ROWS) == half

    def body(a_ref, b_ref, out_ref,
             acc_ref, xrecv_ref,
             x_send_sems, x_recv_sems, y_send_sems, y_recv_sems):
        my_x = lax.axis_index("x")
        my_y = lax.axis_index("y")
        x_nbr = (1 - my_x, my_y)
        y_nbr = (my_x, 1 - my_y)

        my_base = my_y * half

        barrier_sem = pltpu.get_barrier_semaphore()
        for nbr in (x_nbr, y_nbr):
            pl.semaphore_signal(
                barrier_sem, inc=1, device_id=nbr,
                device_id_type=pl.DeviceIdType.MESH,
            )

        acc_ref[:, :] = jnp.dot(
            a_ref[pl.ds(my_base, half), :], b_ref[:, :],
            preferred_element_type=jnp.float32,
        )

        pl.semaphore_wait(barrier_sem, 2)

        x_rdmas = []
        for c in range(N_CHUNKS):
            rows = pl.ds(CHUNK_OFFS[c], CHUNK_ROWS[c])
            rdma = pltpu.make_async_remote_copy(
                src_ref=acc_ref.at[rows, :],
                dst_ref=xrecv_ref.at[rows, :],
                send_sem=x_send_sems.at[c],
                recv_sem=x_recv_sems.at[c],
                device_id=x_nbr,
                device_id_type=pl.DeviceIdType.MESH,
            )
            rdma.start()
            x_rdmas.append(rdma)

        y_rdmas = []
        for c in range(N_CHUNKS):
            rows = pl.ds(CHUNK_OFFS[c], CHUNK_ROWS[c])
            out_rows = pl.ds(my_base + CHUNK_OFFS[c], CHUNK_ROWS[c])
            x_rdmas[c].wait_recv()
            out_ref[out_rows, :] = acc_ref[rows, :] + xrecv_ref[rows, :]
            rdma = pltpu.make_async_remote_copy(
                src_ref=out_ref.at[out_rows, :],
                dst_ref=out_ref.at[out_rows, :],
                send_sem=y_send_sems.at[c],
                recv_sem=y_recv_sems.at[c],
                device_id=y_nbr,
                device_id_type=pl.DeviceIdType.MESH,
            )
            rdma.start()
            y_rdmas.append(rdma)

        for c in range(N_CHUNKS):
            y_rdmas[c].wait_recv()
        for c in range(N_CHUNKS):
            x_rdmas[c].wait_send()
            y_rdmas[c].wait_send()

    return pl.pallas_call(
        body,
        out_shape=jax.ShapeDtypeStruct((m, n), jnp.float32),
        in_specs=[
            pl.BlockSpec(memory_space=pltpu.VMEM),
            pl.BlockSpec(memory_space=pltpu.VMEM),
        ],
        out_specs=pl.BlockSpec(memory_space=pltpu.VMEM),
        scratch_shapes=[
            pltpu.VMEM((half, n), jnp.float32),
            pltpu.VMEM((half, n), jnp.float32),
            pltpu.SemaphoreType.DMA((N_CHUNKS,)),
            pltpu.SemaphoreType.DMA((N_CHUNKS,)),
            pltpu.SemaphoreType.DMA((N_CHUNKS,)),
            pltpu.SemaphoreType.DMA((N_CHUNKS,)),
        ],
        compiler_params=pltpu.CompilerParams(collective_id=0),
    )(A, B)
